# speedup vs baseline: 4.5913x; 4.5913x over previous
import functools

import jax
import jax.numpy as jnp
from jax import lax
from jax.experimental import pallas as pl
from jax.experimental.pallas import tpu as pltpu
from jax.experimental.pallas import tpu_sc as plsc

H = 128
N = 10000
NP = 10240
E = 320000
N2 = 2000
N2P = 2048
NS = 400
G = 16
C = 10

_NWORK = 32
_CHUNK = 80

_mesh = plsc.VectorSubcoreMesh(core_axis_name="c", subcore_axis_name="s")


def _make_sc_gather(n_idx):
    per_w = n_idx // _NWORK
    n_chunks = per_w // _CHUNK

    @functools.partial(
        pl.kernel,
        out_type=jax.ShapeDtypeStruct((n_idx, H), jnp.float32),
        mesh=_mesh,
        scratch_types=[
            pltpu.VMEM((_CHUNK,), jnp.int32),
            pltpu.VMEM((_CHUNK, H), jnp.float32),
            pltpu.SemaphoreType.DMA,
        ],
    )
    def k(table_hbm, idx_hbm, out_hbm, idx_v, rows_v, sem):
        cid = lax.axis_index("c")
        sid = lax.axis_index("s")
        base = (sid * 2 + cid) * per_w

        def body(i, carry):
            off = base + i * _CHUNK
            pltpu.sync_copy(idx_hbm.at[pl.ds(off, _CHUNK)], idx_v)
            pltpu.async_copy(table_hbm.at[idx_v], rows_v, sem).wait()
            pltpu.sync_copy(rows_v, out_hbm.at[pl.ds(off, _CHUNK), :])
            return carry

        lax.fori_loop(0, n_chunks, body, 0)

    return k


def _make_sc_edge_agg():
    per_w = E // _NWORK
    n_chunks = per_w // _CHUNK
    rpt = NP // 16

    @functools.partial(
        pl.kernel,
        out_type=jax.ShapeDtypeStruct((2, NP, H), jnp.float32),
        mesh=_mesh,
        scratch_types=[
            pltpu.VMEM((_CHUNK,), jnp.int32),
            pltpu.VMEM((_CHUNK,), jnp.int32),
            pltpu.VMEM((_CHUNK, H), jnp.float32),
            pltpu.VMEM_SHARED((NP, H), jnp.float32),
            pltpu.SemaphoreType.DMA,
        ],
    )
    def k(m_hbm, src_hbm, dst_hbm, zero_hbm, out_hbm, src_v, dst_v, rows_v,
          acc, sem):
        cid = lax.axis_index("c")
        sid = lax.axis_index("s")
        r0 = sid * rpt
        pltpu.sync_copy(zero_hbm.at[pl.ds(r0, rpt), :],
                        acc.at[pl.ds(r0, rpt), :])
        plsc.subcore_barrier()

        base = (sid * 2 + cid) * per_w

        def body(i, carry):
            off = base + i * _CHUNK
            pltpu.sync_copy(src_hbm.at[pl.ds(off, _CHUNK)], src_v)
            pltpu.sync_copy(dst_hbm.at[pl.ds(off, _CHUNK)], dst_v)
            pltpu.async_copy(m_hbm.at[src_v], rows_v, sem).wait()
            pltpu.sync_copy(rows_v, acc.at[dst_v], add=True)
            return carry

        lax.fori_loop(0, n_chunks, body, 0)
        plsc.subcore_barrier()
        pltpu.sync_copy(acc.at[pl.ds(r0, rpt), :],
                        out_hbm.at[cid, pl.ds(r0, rpt), :])

    return k


def _make_sc_segsum(n_rows, n_seg):
    per_w = n_rows // _NWORK
    n_chunks = per_w // _CHUNK
    rpt = n_seg // 16

    @functools.partial(
        pl.kernel,
        out_type=jax.ShapeDtypeStruct((2, n_seg, H), jnp.float32),
        mesh=_mesh,
        scratch_types=[
            pltpu.VMEM((_CHUNK,), jnp.int32),
            pltpu.VMEM((_CHUNK, H), jnp.float32),
            pltpu.VMEM_SHARED((n_seg, H), jnp.float32),
            pltpu.SemaphoreType.DMA,
        ],
    )
    def k(x_hbm, idx_hbm, zero_hbm, out_hbm, idx_v, rows_v, acc, sem):
        cid = lax.axis_index("c")
        sid = lax.axis_index("s")
        r0 = sid * rpt
        pltpu.sync_copy(zero_hbm.at[pl.ds(r0, rpt), :],
                        acc.at[pl.ds(r0, rpt), :])
        plsc.subcore_barrier()

        base = (sid * 2 + cid) * per_w

        def body(i, carry):
            off = base + i * _CHUNK
            pltpu.sync_copy(idx_hbm.at[pl.ds(off, _CHUNK)], idx_v)
            pltpu.sync_copy(x_hbm.at[pl.ds(off, _CHUNK), :], rows_v)
            pltpu.sync_copy(rows_v, acc.at[idx_v], add=True)
            return carry

        lax.fori_loop(0, n_chunks, body, 0)
        plsc.subcore_barrier()
        pltpu.sync_copy(acc.at[pl.ds(r0, rpt), :],
                        out_hbm.at[cid, pl.ds(r0, rpt), :])

    return k


_sc_emb_gather = _make_sc_gather(NP)
_sc_edge_agg = _make_sc_edge_agg()
_sc_segsum = _make_sc_segsum(NP, N2P)


_BN = 1024


def _tc_relu_mm(zraw, w):
    def body(z_ref, w_ref, zf_ref, m_ref):
        zf = jnp.maximum(z_ref[...], 0.0)
        zf_ref[...] = zf
        m_ref[...] = jnp.dot(zf, w_ref[...], preferred_element_type=jnp.float32)

    return pl.pallas_call(
        body,
        grid=(NP // _BN,),
        in_specs=[pl.BlockSpec((_BN, H), lambda i: (i, 0)),
                  pl.BlockSpec((H, H), lambda i: (0, 0))],
        out_specs=[pl.BlockSpec((_BN, H), lambda i: (i, 0)),
                   pl.BlockSpec((_BN, H), lambda i: (i, 0))],
        out_shape=[jax.ShapeDtypeStruct((NP, H), jnp.float32),
                   jax.ShapeDtypeStruct((NP, H), jnp.float32)],
    )(zraw, w)


def _tc_mm(zf, w):
    def body(z_ref, w_ref, m_ref):
        m_ref[...] = jnp.dot(z_ref[...], w_ref[...],
                             preferred_element_type=jnp.float32)

    return pl.pallas_call(
        body,
        grid=(NP // _BN,),
        in_specs=[pl.BlockSpec((_BN, H), lambda i: (i, 0)),
                  pl.BlockSpec((H, H), lambda i: (0, 0))],
        out_specs=pl.BlockSpec((_BN, H), lambda i: (i, 0)),
        out_shape=jax.ShapeDtypeStruct((NP, H), jnp.float32),
    )(zf, w)


def _tc_gru(aggp, h, wihT, whhT, bih, bhh):
    def body(p_ref, h_ref, wih_ref, whh_ref, bih_ref, bhh_ref, o_ref):
        agg = p_ref[0] + p_ref[1]
        hh = h_ref[...]
        gi = jnp.dot(agg, wih_ref[...],
                     preferred_element_type=jnp.float32) + bih_ref[...]
        gh = jnp.dot(hh, whh_ref[...],
                     preferred_element_type=jnp.float32) + bhh_ref[...]
        r = jax.nn.sigmoid(gi[:, :H] + gh[:, :H])
        u = jax.nn.sigmoid(gi[:, H:2 * H] + gh[:, H:2 * H])
        nn_ = jnp.tanh(gi[:, 2 * H:] + r * gh[:, 2 * H:])
        o_ref[...] = jnp.maximum((1.0 - u) * nn_ + u * hh, 0.0)

    return pl.pallas_call(
        body,
        grid=(NP // _BN,),
        in_specs=[pl.BlockSpec((2, _BN, H), lambda i: (0, i, 0)),
                  pl.BlockSpec((_BN, H), lambda i: (i, 0)),
                  pl.BlockSpec((H, 3 * H), lambda i: (0, 0)),
                  pl.BlockSpec((H, 3 * H), lambda i: (0, 0)),
                  pl.BlockSpec((1, 3 * H), lambda i: (0, 0)),
                  pl.BlockSpec((1, 3 * H), lambda i: (0, 0))],
        out_specs=pl.BlockSpec((_BN, H), lambda i: (i, 0)),
        out_shape=jax.ShapeDtypeStruct((NP, H), jnp.float32),
    )(aggp, h, wihT, whhT, bih, bhh)


def _tc_hier(nsp, s2s_pad, s2g, x, pxW, pxb, ew1, eb1, ew2, eb2,
             nw1, nb1, nw2, nb2):
    def body(p_ref, s2s_ref, s2g_ref, x_ref, pxw_ref, pxb_ref,
             ew1_ref, eb1_ref, ew2_ref, eb2_ref,
             nw1_ref, nb1_ref, nw2_ref, nb2_ref, o_ref):
        ne = p_ref[0] + p_ref[1]
        h1 = jnp.maximum(
            jnp.dot(ne, ew1_ref[...], preferred_element_type=jnp.float32)
            + eb1_ref[...], 0.0)
        ne2 = jnp.dot(h1, ew2_ref[...],
                      preferred_element_type=jnp.float32) + eb2_ref[...]
        oh1 = (lax.broadcasted_iota(jnp.int32, (NS, N2P), 0)
               == s2s_ref[...]).astype(jnp.float32)
        sub = jnp.dot(oh1, ne2, preferred_element_type=jnp.float32)
        h2 = jnp.maximum(
            jnp.dot(sub, nw1_ref[...], preferred_element_type=jnp.float32)
            + nb1_ref[...], 0.0)
        sub2 = jnp.dot(h2, nw2_ref[...],
                       preferred_element_type=jnp.float32) + nb2_ref[...]
        xf = jnp.maximum(
            jnp.dot(x_ref[...], pxw_ref[...],
                    preferred_element_type=jnp.float32) + pxb_ref[...], 0.0)
        oh2 = (lax.broadcasted_iota(jnp.int32, (G, NS), 0)
               == s2g_ref[...]).astype(jnp.float32)
        o_ref[...] = jnp.dot(oh2, sub2 * xf,
                             preferred_element_type=jnp.float32)

    return pl.pallas_call(
        body,
        out_shape=jax.ShapeDtypeStruct((G, H), jnp.float32),
    )(nsp, s2s_pad, s2g, x, pxW, pxb, ew1, eb1, ew2, eb2, nw1, nb1, nw2, nb2)


def _tc_post(ges, w1, b1, w2p, b2p):
    def body(g_ref, w1_ref, b1_ref, w2_ref, b2_ref, o_ref):
        e = g_ref[0] + g_ref[1] + g_ref[2]
        hh = jnp.maximum(
            jnp.dot(e, w1_ref[...], preferred_element_type=jnp.float32)
            + b1_ref[...], 0.0)
        logits = jnp.dot(hh, w2_ref[...],
                         preferred_element_type=jnp.float32) + b2_ref[...]
        mx = jnp.max(logits, axis=1, keepdims=True)
        lse = jnp.log(jnp.sum(jnp.exp(logits - mx), axis=1,
                              keepdims=True)) + mx
        o_ref[...] = logits - lse

    return pl.pallas_call(
        body,
        out_shape=jax.ShapeDtypeStruct((G, H), jnp.float32),
    )(ges, w1, b1, w2p, b2p)


def kernel(z, x, edge_index, batch, node_to_subgraph2, subgraph2_to_subgraph,
           subgraph_to_graph,
           emb, pxW, pxb, ie_w1, ie_b1, ie_w2, ie_b2,
           in_w1, in_b1, in_w2, in_b2,
           conv0_w, conv0_wih, conv0_whh, conv0_bih, conv0_bhh,
           e0_w1, e0_b1, e0_w2, e0_b2, n0_w1, n0_b1, n0_w2, n0_b2,
           conv1_w, conv1_wih, conv1_whh, conv1_bih, conv1_bhh,
           e1_w1, e1_b1, e1_w2, e1_b2, n1_w1, n1_b1, n1_w2, n1_b2,
           post_w1, post_b1, post_w2, post_b2):
    i32 = jnp.int32
    z_pad = jnp.concatenate([z.astype(i32), jnp.zeros((NP - N,), i32)])
    n2s2_pad = jnp.concatenate([node_to_subgraph2.astype(i32),
                                jnp.full((NP - N,), N2P - 1, i32)])
    s2s_pad = jnp.concatenate([subgraph2_to_subgraph.astype(i32),
                               jnp.full((N2P - N2,), NS, i32)]).reshape(1, N2P)
    s2g = subgraph_to_graph.astype(i32).reshape(1, NS)
    src = edge_index[0].astype(i32)
    dst = edge_index[1].astype(i32)
    zero_np = jnp.zeros((NP, H), jnp.float32)
    zero_n2 = jnp.zeros((N2P, H), jnp.float32)

    r = lambda b: b.reshape(1, -1)
    w2p = jnp.concatenate([post_w2, jnp.zeros((H, H - C), jnp.float32)], 1)
    b2p = jnp.concatenate([post_b2,
                           jnp.full((H - C,), -1e30, jnp.float32)]).reshape(1, H)

    zraw = _sc_emb_gather(emb, z_pad)
    zf0, m0 = _tc_relu_mm(zraw, conv0_w)

    aggp0 = _sc_edge_agg(m0, src, dst, zero_np)
    zf1 = _tc_gru(aggp0, zf0, conv0_wih.T, conv0_whh.T,
                  r(conv0_bih), r(conv0_bhh))
    m1 = _tc_mm(zf1, conv1_w)
    aggp1 = _sc_edge_agg(m1, src, dst, zero_np)
    zf2 = _tc_gru(aggp1, zf1, conv1_wih.T, conv1_whh.T,
                  r(conv1_bih), r(conv1_bhh))

    ges = []
    for zf, ws in ((zf0, (ie_w1, ie_b1, ie_w2, ie_b2,
                          in_w1, in_b1, in_w2, in_b2)),
                   (zf1, (e0_w1, e0_b1, e0_w2, e0_b2,
                          n0_w1, n0_b1, n0_w2, n0_b2)),
                   (zf2, (e1_w1, e1_b1, e1_w2, e1_b2,
                          n1_w1, n1_b1, n1_w2, n1_b2))):
        w1, b1, w2, b2, v1, c1, v2, c2 = ws
        nsp = _sc_segsum(zf, n2s2_pad, zero_n2)
        ges.append(_tc_hier(nsp, s2s_pad, s2g, x, pxW, r(pxb),
                            w1, r(b1), w2, r(b2), v1, r(c1), v2, r(c2)))

    out = _tc_post(jnp.stack(ges), post_w1, r(post_b1), w2p, b2p)
    return out[:, :C]
